# baseline (speedup 1.0000x reference)
import functools

import jax
import jax.numpy as jnp
from jax import lax
from jax.experimental import pallas as pl
from jax.experimental.pallas import tpu as pltpu
from jax.experimental.pallas import tpu_sc as plsc

NUM_TOKENS = 200
EMB_DIM = 4096
BATCH = 128

NUM_CORES = 2
NUM_SUBCORES = 16
NUM_WORKERS = NUM_CORES * NUM_SUBCORES
BATCHES_PER_WORKER = BATCH // NUM_WORKERS

CHUNK_ROWS = 8
NUM_CHUNKS = NUM_TOKENS // CHUNK_ROWS
NUM_SLOTS = 3


@functools.partial(
    pl.kernel,
    mesh=plsc.VectorSubcoreMesh(core_axis_name="c", subcore_axis_name="s"),
    out_type=jax.ShapeDtypeStruct((BATCH, NUM_TOKENS, EMB_DIM), jnp.float32),
    scratch_types=(
        [pltpu.VMEM((CHUNK_ROWS, EMB_DIM), jnp.float32)] * NUM_SLOTS
        + [pltpu.SemaphoreType.DMA] * (2 * NUM_SLOTS)
    ),
)
def _broadcast_table(table_hbm, out_hbm, *scratch):
    bufs = scratch[:NUM_SLOTS]
    lsems = scratch[NUM_SLOTS : 2 * NUM_SLOTS]
    wsems = scratch[2 * NUM_SLOTS :]

    wid = lax.axis_index("s") * NUM_CORES + lax.axis_index("c")
    batch0 = wid * BATCHES_PER_WORKER

    def start_load(c):
        s = c % NUM_SLOTS
        return pltpu.async_copy(
            table_hbm.at[pl.ds(CHUNK_ROWS * c, CHUNK_ROWS)], bufs[s], lsems[s]
        )

    loads = {c: start_load(c) for c in range(NUM_SLOTS)}
    writes = {}
    for c in range(NUM_CHUNKS):
        s = c % NUM_SLOTS
        loads[c].wait()
        writes[c] = [
            pltpu.async_copy(
                bufs[s],
                out_hbm.at[batch0 + i, pl.ds(CHUNK_ROWS * c, CHUNK_ROWS)],
                wsems[s],
            )
            for i in range(BATCHES_PER_WORKER)
        ]
        n = c + NUM_SLOTS - 1
        if NUM_SLOTS <= n < NUM_CHUNKS:
            for h in writes.pop(n - NUM_SLOTS):
                h.wait()
            loads[n] = start_load(n)
    for c in sorted(writes):
        for h in writes[c]:
            h.wait()


def kernel(batch_size, virtual_embeddings):
    scale = (jnp.asarray(batch_size, jnp.int32) - BATCH + 1).astype(
        virtual_embeddings.dtype
    )
    return _broadcast_table(virtual_embeddings * scale)
